# baseline (speedup 1.0000x reference)
import functools

import jax
import jax.numpy as jnp
from jax import lax
from jax.experimental import pallas as pl
from jax.experimental.pallas import tpu as pltpu
from jax.experimental.pallas import tpu_sc as plsc

_B = 32
_K = 1024
_DIM = 50257
_KNN = 8
_NCLASS = 2
_INV_T = 20.0

_DB = 1024
_ND = -(-_DIM // _DB)

_LN_POLY = (1.024394859e-02, -5.326809416e-02, 1.319908369e-01,
            -2.239679875e-01, 3.275122369e-01, -4.993340753e-01,
            9.999702565e-01, 2.212031243e-07)
_LN2 = 0.6931471805599453


def _fast_ln(x):
    bits = lax.bitcast_convert_type(x, jnp.int32)
    e = ((bits >> 23) - 127).astype(jnp.float32)
    m = lax.bitcast_convert_type(
        (bits & 0x007FFFFF) | 0x3F800000, jnp.float32)
    t = m - 1.0
    p = jnp.full_like(t, _LN_POLY[0])
    for c in _LN_POLY[1:]:
        p = p * t + c
    return e * _LN2 + p


def _dist_body(q_ref, at_ref, out_ref, accc, accl, acca):
    di = pl.program_id(0)
    nd = pl.num_programs(0)

    @pl.when(di == 0)
    def _init():
        accc[...] = jnp.zeros_like(accc)
        accl[...] = jnp.zeros_like(accl)
        acca[...] = jnp.zeros_like(acca)

    dmask_r = di * _DB + lax.broadcasted_iota(jnp.int32, (_DB, 1), 0) < _DIM
    dmask_c = di * _DB + lax.broadcasted_iota(jnp.int32, (1, _DB), 1) < _DIM
    at = jnp.where(dmask_r, at_ref[...], 1.0)
    la = jnp.log(at)
    lq = jnp.log(jnp.where(dmask_c, q_ref[...], 1.0))

    def _hi(x):
        return x.astype(jnp.bfloat16).astype(jnp.float32)

    at_hi = _hi(at)
    at_lo = at - at_hi
    lq_hi = _hi(lq)
    lq_lo = lq - lq_hi

    lhs = jnp.concatenate([lq_hi, lq_lo, lq_hi], axis=1).astype(jnp.bfloat16)
    rhs = jnp.concatenate([at_hi, at_hi, at_lo], axis=0).astype(jnp.bfloat16)
    dn = (((1,), (0,)), ((), ()))
    accc[...] += lax.dot_general(lhs, rhs, dn, preferred_element_type=jnp.float32)
    acca[...] += jnp.sum(at * la, axis=0, keepdims=True)

    @pl.when(di == nd - 1)
    def _fin():
        out_ref[...] = (_INV_T / _DIM) * (accc[...] + accl[...] - acca[...])


def _scaled_dists(logits, queue_anchor):
    a_t = queue_anchor.T
    return pl.pallas_call(
        _dist_body,
        grid=(_ND,),
        in_specs=[
            pl.BlockSpec((_B, _DB), lambda di: (0, di)),
            pl.BlockSpec((_DB, _K), lambda di: (di, 0)),
        ],
        out_specs=pl.BlockSpec((_B, _K), lambda di: (0, 0)),
        out_shape=jax.ShapeDtypeStruct((_B, _K), jnp.float32),
        scratch_shapes=[
            pltpu.VMEM((_B, _K), jnp.float32),
            pltpu.VMEM((_B, _K), jnp.float32),
            pltpu.VMEM((1, _K), jnp.float32),
        ],
        compiler_params=pltpu.CompilerParams(
            dimension_semantics=("arbitrary",),
            vmem_limit_bytes=100 * 1024 * 1024,
        ),
    )(logits, a_t)


def _knn_tail(scaled, queue_label):
    info = plsc.get_sparse_core_info()
    nc, ns = info.num_cores, info.num_subcores
    assert nc * ns == _B

    mesh = plsc.VectorSubcoreMesh(core_axis_name="c", subcore_axis_name="s")

    @functools.partial(
        pl.kernel,
        mesh=mesh,
        out_type=jax.ShapeDtypeStruct((_B, 16), jnp.float32),
        scratch_types=[
            pltpu.VMEM((_K,), jnp.float32),
            pltpu.VMEM((_K,), jnp.int32),
            pltpu.VMEM((16,), jnp.float32),
        ],
        compiler_params=pltpu.CompilerParams(needs_layout_passes=False),
    )
    def tail(scaled_hbm, label_hbm, out_hbm, row_v, lab_v, out_v):
        b = lax.axis_index("s") * nc + lax.axis_index("c")
        pltpu.sync_copy(scaled_hbm.at[b], row_v)
        pltpu.sync_copy(label_hbm, lab_v)

        lane = lax.iota(jnp.int32, 16)
        rk = jnp.full((16,), -3.4e38, jnp.float32)
        rl = jnp.zeros((16,), jnp.int32)
        for c in range(_K // 16):
            ck = row_v[pl.ds(c * 16, 16)]
            cl = lab_v[pl.ds(c * 16, 16)]
            ck_s, cl_s = plsc.sort_key_val(ck, cl, descending=True)
            take_r = rk >= ck_s
            nk = jnp.where(take_r, rk, ck_s)
            nl = jnp.where(take_r, rl, cl_s)
            rk, rl = plsc.sort_key_val(nk, nl, descending=False)

        top8 = lane >= 8
        m = jnp.max(rk)
        w = jnp.where(top8, jnp.exp(rk - m), 0.0)
        s1 = jnp.sum(jnp.where(rl == 1, w, 0.0))
        s0 = jnp.sum(jnp.where(rl == 0, w, 0.0))
        d = jnp.full((16,), s0 + s1, jnp.float32)
        r = lax.bitcast_convert_type(
            jnp.full((16,), 0x7EF127EA, jnp.int32)
            - lax.bitcast_convert_type(d, jnp.int32), jnp.float32)
        for _ in range(3):
            r = r * (2.0 - d * r)
        out_v[...] = jnp.where(lane == 0, s0, jnp.where(lane == 1, s1, 0.0)) * r
        pltpu.sync_copy(out_v, out_hbm.at[b])

    return tail(scaled, queue_label)


def kernel(logits, queue_anchor, queue_label):
    scaled = _scaled_dists(logits, queue_anchor)
    out16 = _knn_tail(scaled, queue_label)
    return out16[:, :_NCLASS]


# speedup vs baseline: 1.2407x; 1.2407x over previous
import functools

import jax
import jax.numpy as jnp
from jax import lax
from jax.experimental import pallas as pl
from jax.experimental.pallas import tpu as pltpu
from jax.experimental.pallas import tpu_sc as plsc

_B = 32
_K = 1024
_DIM = 50257
_KNN = 8
_NCLASS = 2
_INV_T = 20.0

_DB = 2048
_ND = -(-_DIM // _DB)

_LN_POLY = (1.024394859e-02, -5.326809416e-02, 1.319908369e-01,
            -2.239679875e-01, 3.275122369e-01, -4.993340753e-01,
            9.999702565e-01, 2.212031243e-07)
_LN2 = 0.6931471805599453


def _fast_ln(x):
    bits = lax.bitcast_convert_type(x, jnp.int32)
    e = ((bits >> 23) - 127).astype(jnp.float32)
    m = lax.bitcast_convert_type(
        (bits & 0x007FFFFF) | 0x3F800000, jnp.float32)
    t = m - 1.0
    p = jnp.full_like(t, _LN_POLY[0])
    for c in _LN_POLY[1:]:
        p = p * t + c
    return e * _LN2 + p


def _accumulate(at, lq, accc, acca):
    la = jnp.log(at)

    def _hi(x):
        return x.astype(jnp.bfloat16).astype(jnp.float32)

    at_hi = _hi(at)
    at_lo = at - at_hi
    lq_hi = _hi(lq)
    lq_lo = lq - lq_hi
    lhs = jnp.concatenate([lq_hi, lq_lo, lq_hi], axis=1).astype(jnp.bfloat16)
    rhs = jnp.concatenate([at_hi, at_hi, at_lo], axis=0).astype(jnp.bfloat16)
    dn = (((1,), (0,)), ((), ()))
    accc[...] += lax.dot_general(lhs, rhs, dn, preferred_element_type=jnp.float32)
    acca[...] += jnp.sum(at * la, axis=0, keepdims=True)


def _dist_body(q_ref, at_ref, out_ref, accc, acca):
    di = pl.program_id(0)
    nd = pl.num_programs(0)

    @pl.when(di == 0)
    def _init():
        accc[...] = jnp.zeros_like(accc)
        acca[...] = jnp.zeros_like(acca)

    @pl.when(di < nd - 1)
    def _steady():
        _accumulate(at_ref[...], jnp.log(q_ref[...]), accc, acca)

    @pl.when(di == nd - 1)
    def _last():
        dmask_r = di * _DB + lax.broadcasted_iota(jnp.int32, (_DB, 1), 0) < _DIM
        dmask_c = di * _DB + lax.broadcasted_iota(jnp.int32, (1, _DB), 1) < _DIM
        at = jnp.where(dmask_r, at_ref[...], 1.0)
        lq = jnp.log(jnp.where(dmask_c, q_ref[...], 1.0))
        _accumulate(at, lq, accc, acca)
        out_ref[...] = (_INV_T / _DIM) * (accc[...] - acca[...])


def _scaled_dists(logits, queue_anchor):
    a_t = queue_anchor.T
    return pl.pallas_call(
        _dist_body,
        grid=(_ND,),
        in_specs=[
            pl.BlockSpec((_B, _DB), lambda di: (0, di)),
            pl.BlockSpec((_DB, _K), lambda di: (di, 0)),
        ],
        out_specs=pl.BlockSpec((_B, _K), lambda di: (0, 0)),
        out_shape=jax.ShapeDtypeStruct((_B, _K), jnp.float32),
        scratch_shapes=[
            pltpu.VMEM((_B, _K), jnp.float32),
            pltpu.VMEM((1, _K), jnp.float32),
        ],
        compiler_params=pltpu.CompilerParams(
            dimension_semantics=("arbitrary",),
            vmem_limit_bytes=100 * 1024 * 1024,
        ),
    )(logits, a_t)


def _knn_tail(scaled, queue_label):
    info = plsc.get_sparse_core_info()
    nc, ns = info.num_cores, info.num_subcores
    assert nc * ns == _B

    mesh = plsc.VectorSubcoreMesh(core_axis_name="c", subcore_axis_name="s")

    @functools.partial(
        pl.kernel,
        mesh=mesh,
        out_type=jax.ShapeDtypeStruct((_B, 16), jnp.float32),
        scratch_types=[
            pltpu.VMEM((_K,), jnp.float32),
            pltpu.VMEM((_K,), jnp.int32),
            pltpu.VMEM((16,), jnp.float32),
        ],
        compiler_params=pltpu.CompilerParams(needs_layout_passes=False),
    )
    def tail(scaled_hbm, label_hbm, out_hbm, row_v, lab_v, out_v):
        b = lax.axis_index("s") * nc + lax.axis_index("c")
        pltpu.sync_copy(scaled_hbm.at[b], row_v)
        pltpu.sync_copy(label_hbm, lab_v)

        lane = lax.iota(jnp.int32, 16)
        rk = jnp.full((16,), -3.4e38, jnp.float32)
        rl = jnp.zeros((16,), jnp.int32)
        for c in range(_K // 16):
            ck = row_v[pl.ds(c * 16, 16)]
            cl = lab_v[pl.ds(c * 16, 16)]
            ck_s, cl_s = plsc.sort_key_val(ck, cl, descending=True)
            take_r = rk >= ck_s
            nk = jnp.where(take_r, rk, ck_s)
            nl = jnp.where(take_r, rl, cl_s)
            rk, rl = plsc.sort_key_val(nk, nl, descending=False)

        top8 = lane >= 8
        m = jnp.max(rk)
        w = jnp.where(top8, jnp.exp(rk - m), 0.0)
        s1 = jnp.sum(jnp.where(rl == 1, w, 0.0))
        s0 = jnp.sum(jnp.where(rl == 0, w, 0.0))
        d = jnp.full((16,), s0 + s1, jnp.float32)
        r = lax.bitcast_convert_type(
            jnp.full((16,), 0x7EF127EA, jnp.int32)
            - lax.bitcast_convert_type(d, jnp.int32), jnp.float32)
        for _ in range(3):
            r = r * (2.0 - d * r)
        out_v[...] = jnp.where(lane == 0, s0, jnp.where(lane == 1, s1, 0.0)) * r
        pltpu.sync_copy(out_v, out_hbm.at[b])

    return tail(scaled, queue_label)


def kernel(logits, queue_anchor, queue_label):
    scaled = _scaled_dists(logits, queue_anchor)
    out16 = _knn_tail(scaled, queue_label)
    return out16[:, :_NCLASS]
